# baseline (speedup 1.0000x reference)
import functools

import jax
import jax.numpy as jnp
from jax import lax
from jax.experimental import pallas as pl
from jax.experimental.pallas import tpu as pltpu
from jax.experimental.pallas import tpu_sc as plsc

N = 10000
NPAD = 10240
E = 320000
DIN = 128
DOUT = 128
CHUNK = 128
EDGES_PER_CORE = E // 2
NUM_CHUNKS = EDGES_PER_CORE // CHUNK
NS = 16
ROWS_PER_TILE = NPAD // NS
RB = 128


def _mm_body(x_ref, w_ref, o_ref):
    o_ref[...] = jnp.dot(x_ref[...], w_ref[...], preferred_element_type=jnp.float32)


def _matmul(x, W):
    bm = 1000
    return pl.pallas_call(
        _mm_body,
        grid=(N // bm,),
        in_specs=[
            pl.BlockSpec((bm, DIN), lambda i: (i, 0)),
            pl.BlockSpec((DIN, DOUT), lambda i: (0, 0)),
        ],
        out_specs=pl.BlockSpec((bm, DOUT), lambda i: (i, 0)),
        out_shape=jax.ShapeDtypeStruct((N, DOUT), jnp.float32),
    )(x, W)


def _combine_body(p_ref, o_ref):
    o_ref[...] = jnp.maximum(p_ref[0] + p_ref[1], 0.0)


def _combine_relu(partials):
    bm = 1000
    return pl.pallas_call(
        _combine_body,
        grid=(N // bm,),
        in_specs=[pl.BlockSpec((2, bm, DOUT), lambda i: (0, i, 0))],
        out_specs=pl.BlockSpec((bm, DOUT), lambda i: (i, 0)),
        out_shape=jax.ShapeDtypeStruct((N, DOUT), jnp.float32),
    )(partials)


@functools.partial(
    pl.kernel,
    out_type=jax.ShapeDtypeStruct((2, NPAD, DOUT), jnp.float32),
    mesh=plsc.VectorSubcoreMesh(core_axis_name="c", subcore_axis_name="s"),
    scratch_types=[
        pltpu.VMEM((2, CHUNK), jnp.int32),
        pltpu.VMEM((2, CHUNK), jnp.int32),
        pltpu.VMEM((2, CHUNK), jnp.float32),
        pltpu.VMEM((2, CHUNK, DOUT), jnp.float32),
        pltpu.VMEM_SHARED((NPAD, DOUT), jnp.float32),
        pltpu.SemaphoreType.DMA((2,)),
        pltpu.SemaphoreType.DMA((2,)),
    ],
)
def _sc_aggregate(pre_hbm, src_hbm, dst_hbm, ew_hbm, out_hbm,
                  src_v, dst_v, ew_v, rows_v, acc, sem_i, sem_g):
    c = lax.axis_index("c")
    s = lax.axis_index("s")
    row0 = s * ROWS_PER_TILE
    n = (NUM_CHUNKS - s + NS - 1) // NS

    def _idx_copies(g, p):
        e0 = (c * NUM_CHUNKS + g) * CHUNK
        return (
            pltpu.make_async_copy(src_hbm.at[pl.ds(e0, CHUNK)], src_v.at[p], sem_i.at[p]),
            pltpu.make_async_copy(dst_hbm.at[pl.ds(e0, CHUNK)], dst_v.at[p], sem_i.at[p]),
            pltpu.make_async_copy(ew_hbm.at[pl.ds(e0, CHUNK)], ew_v.at[p], sem_i.at[p]),
        )

    def _start_idx(g, p):
        for cp in _idx_copies(g, p):
            cp.start()

    def _wait_idx(g, p):
        for cp in _idx_copies(g, p):
            cp.wait()

    def _gather(p):
        return pltpu.make_async_copy(pre_hbm.at[src_v.at[p]], rows_v.at[p], sem_g.at[p])

    def _zero_row(r, carry):
        for j in range(DOUT // 16):
            rows_v[0, r, pl.ds(j * 16, 16)] = jnp.zeros((16,), jnp.float32)
        return carry

    lax.fori_loop(0, RB, _zero_row, 0)
    for b in range(ROWS_PER_TILE // RB):
        pltpu.sync_copy(rows_v.at[0, pl.ds(0, RB)],
                        acc.at[pl.ds(row0 + b * RB, RB)])
    plsc.subcore_barrier()

    _start_idx(s, 0)
    _wait_idx(s, 0)
    _gather(0).start()
    _start_idx(s + NS, 1)

    def _chunk(i, carry):
        p = lax.rem(i, 2)
        q = 1 - p
        g = s + i * NS

        @pl.when(i + 1 < n)
        def _():
            _wait_idx(g + NS, q)
            _gather(q).start()

        _gather(p).wait()

        def _scale(eg, carry2):
            w16 = ew_v[p, pl.ds(eg * 16, 16)]
            for k in range(16):
                e = eg * 16 + k
                wk = w16[k]
                for j in range(DOUT // 16):
                    sl = pl.ds(j * 16, 16)
                    rows_v[p, e, sl] = rows_v[p, e, sl] * wk
            return carry2

        lax.fori_loop(0, CHUNK // 16, _scale, 0)
        pltpu.sync_copy(rows_v.at[p], acc.at[dst_v.at[p]], add=True)

        @pl.when(i + 2 < n)
        def _():
            _start_idx(g + 2 * NS, p)

        return carry

    lax.fori_loop(0, n, _chunk, 0)
    plsc.subcore_barrier()

    pltpu.sync_copy(acc.at[pl.ds(row0, ROWS_PER_TILE)],
                    out_hbm.at[c, pl.ds(row0, ROWS_PER_TILE)])


def kernel(x, edge_index, edge_weight, W):
    pre = _matmul(x, W)
    partials = _sc_aggregate(pre, edge_index[0], edge_index[1], edge_weight)
    return _combine_relu(partials)


# speedup vs baseline: 1.1395x; 1.1395x over previous
import functools

import jax
import jax.numpy as jnp
from jax import lax
from jax.experimental import pallas as pl
from jax.experimental.pallas import tpu as pltpu
from jax.experimental.pallas import tpu_sc as plsc

N = 10000
NPAD = 10240
E = 320000
DIN = 128
DOUT = 128
CHUNK = 128
EDGES_PER_CORE = E // 2
CPC = 1280
PAD_TAIL = CPC * CHUNK - EDGES_PER_CORE
NS = 16
CPT = CPC // NS
NBT = CPT // 8
BPC = CPC // 8
ROWS_PER_TILE = NPAD // NS
RB = 128


def _mm_body(x_ref, w_ref, o_ref):
    o_ref[...] = jnp.dot(x_ref[...], w_ref[...], preferred_element_type=jnp.float32)


def _matmul(x, W):
    bm = 1000
    return pl.pallas_call(
        _mm_body,
        grid=(N // bm,),
        in_specs=[
            pl.BlockSpec((bm, DIN), lambda i: (i, 0)),
            pl.BlockSpec((DIN, DOUT), lambda i: (0, 0)),
        ],
        out_specs=pl.BlockSpec((bm, DOUT), lambda i: (i, 0)),
        out_shape=jax.ShapeDtypeStruct((N, DOUT), jnp.float32),
    )(x, W)


def _combine_body(p_ref, o_ref):
    o_ref[...] = jnp.maximum(p_ref[0] + p_ref[1], 0.0)


def _combine_relu(partials):
    bm = 1000
    return pl.pallas_call(
        _combine_body,
        grid=(N // bm,),
        in_specs=[pl.BlockSpec((2, bm, DOUT), lambda i: (0, i, 0))],
        out_specs=pl.BlockSpec((bm, DOUT), lambda i: (i, 0)),
        out_shape=jax.ShapeDtypeStruct((N, DOUT), jnp.float32),
    )(partials)


@functools.partial(
    pl.kernel,
    out_type=jax.ShapeDtypeStruct((2, NPAD, DOUT), jnp.float32),
    mesh=plsc.VectorSubcoreMesh(core_axis_name="c", subcore_axis_name="s"),
    scratch_types=[
        pltpu.VMEM((2, 8, CHUNK), jnp.int32),
        pltpu.VMEM((2, 8, CHUNK), jnp.int32),
        pltpu.VMEM((2, 8, CHUNK), jnp.float32),
        pltpu.VMEM((2, CHUNK, DOUT), jnp.float32),
        pltpu.VMEM_SHARED((NPAD, DOUT), jnp.float32),
        pltpu.SemaphoreType.DMA((2,)),
        pltpu.SemaphoreType.DMA((2,)),
        pltpu.SemaphoreType.DMA((2,)),
    ],
)
def _sc_aggregate(pre_hbm, src_hbm, dst_hbm, ew_hbm, out_hbm,
                  src_v, dst_v, ew_v, rows_v, acc, sem_i, sem_g, sem_s):
    c = lax.axis_index("c")
    s = lax.axis_index("s")
    row0 = s * ROWS_PER_TILE

    def _idx_copies(b, pb):
        bg = c * BPC + s * NBT + b
        return (
            pltpu.make_async_copy(src_hbm.at[bg], src_v.at[pb], sem_i.at[pb]),
            pltpu.make_async_copy(dst_hbm.at[bg], dst_v.at[pb], sem_i.at[pb]),
            pltpu.make_async_copy(ew_hbm.at[bg], ew_v.at[pb], sem_i.at[pb]),
        )

    def _start_idx(b, pb):
        for cp in _idx_copies(b, pb):
            cp.start()

    def _wait_idx(b, pb):
        for cp in _idx_copies(b, pb):
            cp.wait()

    def _gth(pb, i, h):
        return pltpu.make_async_copy(
            pre_hbm.at[src_v.at[pb, i]], rows_v.at[h], sem_g.at[h])

    def _sct_start(pb, i, h):
        pltpu.async_copy(
            rows_v.at[h], acc.at[dst_v.at[pb, i]], sem_s.at[h], add=True)

    def _sct_wait(pb, i, h):
        pltpu.make_async_copy(
            rows_v.at[h], acc.at[dst_v.at[pb, i]], sem_s.at[h]).wait()

    def _scale(pb, i, h):
        def body(eg, carry):
            w16 = ew_v[pb, i, pl.ds(eg * 16, 16)]
            for k in range(16):
                e = eg * 16 + k
                wk = w16[k]
                for j in range(DOUT // 16):
                    sl = pl.ds(j * 16, 16)
                    rows_v[h, e, sl] = rows_v[h, e, sl] * wk
            return carry

        lax.fori_loop(0, CHUNK // 16, body, 0)

    def _zero_row(r, carry):
        for j in range(DOUT // 16):
            rows_v[0, r, pl.ds(j * 16, 16)] = jnp.zeros((16,), jnp.float32)
        return carry

    lax.fori_loop(0, RB, _zero_row, 0)
    for b in range(ROWS_PER_TILE // RB):
        pltpu.sync_copy(rows_v.at[0, pl.ds(0, RB)],
                        acc.at[pl.ds(row0 + b * RB, RB)])
    plsc.subcore_barrier()

    _start_idx(0, 0)
    _start_idx(1, 1)
    _wait_idx(0, 0)
    _gth(0, 0, 0).start()

    def _run_batch(t, pb):
        b = 2 * t + pb

        def _chunk_pair(q, carry):
            i = 2 * q
            _gth(pb, i, 0).wait()
            if pb == 0:
                @pl.when(jnp.logical_and(q == 0, t > 0))
                def _():
                    _sct_wait(1, 7, 1)
            else:
                @pl.when(q == 0)
                def _():
                    _sct_wait(0, 7, 1)
            @pl.when(q > 0)
            def _():
                _sct_wait(pb, i - 1, 1)
            _gth(pb, i + 1, 1).start()
            _scale(pb, i, 0)
            _sct_start(pb, i, 0)

            _gth(pb, i + 1, 1).wait()
            _sct_wait(pb, i, 0)
            @pl.when(q < 3)
            def _():
                _gth(pb, i + 2, 0).start()
            if pb == 0:
                @pl.when(q == 3)
                def _():
                    _wait_idx(b + 1, 1)
                    _gth(1, 0, 0).start()
            else:
                @pl.when(jnp.logical_and(q == 3, t < NBT // 2 - 1))
                def _():
                    _wait_idx(b + 1, 0)
                    _gth(0, 0, 0).start()
            _scale(pb, i + 1, 1)
            _sct_start(pb, i + 1, 1)
            return carry

        lax.fori_loop(0, 4, _chunk_pair, 0)

        @pl.when(b + 2 < NBT)
        def _():
            _start_idx(b + 2, pb)

    def _batch_pair(t, carry):
        _run_batch(t, 0)
        _run_batch(t, 1)
        return carry

    lax.fori_loop(0, NBT // 2, _batch_pair, 0)
    _sct_wait(1, 7, 1)
    plsc.subcore_barrier()

    pltpu.sync_copy(acc.at[pl.ds(row0, ROWS_PER_TILE)],
                    out_hbm.at[c, pl.ds(row0, ROWS_PER_TILE)])


def _pad_split(a):
    z = jnp.zeros((PAD_TAIL,), a.dtype)
    return jnp.concatenate(
        [a[:EDGES_PER_CORE], z, a[EDGES_PER_CORE:], z]
    ).reshape(2 * BPC, 8, CHUNK)


def kernel(x, edge_index, edge_weight, W):
    pre = _matmul(x, W)
    partials = _sc_aggregate(
        pre,
        _pad_split(edge_index[0]),
        _pad_split(edge_index[1]),
        _pad_split(edge_weight),
    )
    return _combine_relu(partials)
